# baseline (speedup 1.0000x reference)
import functools

import jax
import jax.numpy as jnp
from jax import lax
from jax.experimental import pallas as pl
from jax.experimental.pallas import tpu as pltpu
from jax.experimental.pallas import tpu_sc as plsc

N = 10000
E = 320000
F = 128
H = 128
C = 10
NG = 128

NC = 2
NS = 16
NW = NC * NS
CHD = 100
NCHD = E // (NW * CHD)
HH = H // NC
CH = 100
NB = 4
PHASES = 2
PCH = E // (NS * PHASES * CH)
RPT = 624
RPT_LAST = N - (NS - 1) * RPT


def _deg_body(col3_hbm, ones_hbm, zeros1_hbm, out_hbm, colv, onesv, acc1):
    c = lax.axis_index("c")
    s = lax.axis_index("s")
    w = c * NS + s
    pltpu.sync_copy(col3_hbm.at[w], colv)
    pltpu.sync_copy(ones_hbm, onesv)

    @pl.when(s == 0)
    def _():
        pltpu.sync_copy(zeros1_hbm, acc1)

    plsc.subcore_barrier()

    def step(j, carry):
        pltpu.sync_copy(onesv.at[pl.ds(0, CHD)], acc1.at[colv.at[j]], add=True)
        return carry

    lax.fori_loop(0, NCHD, step, 0)
    plsc.subcore_barrier()

    @pl.when(s == 0)
    def _():
        pltpu.sync_copy(acc1, out_hbm.at[c])


def _prop_body(hp_hbm, row3_hbm, col3_hbm, zeros_hbm, out_hbm,
               rowv, colv, bufbig, acc, sem0, sem1, sem2, sem3):
    c = lax.axis_index("c")
    s = lax.axis_index("s")
    w = c * NS + s
    r0 = s * RPT

    @pl.when(s < NS - 1)
    def _():
        pltpu.sync_copy(zeros_hbm.at[pl.ds(r0, RPT)], acc.at[pl.ds(r0, RPT)])

    @pl.when(s == NS - 1)
    def _():
        pltpu.sync_copy(zeros_hbm.at[pl.ds((NS - 1) * RPT, RPT_LAST)],
                        acc.at[pl.ds((NS - 1) * RPT, RPT_LAST)])

    plsc.subcore_barrier()

    bufs = tuple(bufbig.at[pl.ds(b * CH, CH)] for b in range(NB))
    sems = (sem0, sem1, sem2, sem3)

    def gather(j, b):
        pltpu.async_copy(hp_hbm.at[rowv.at[j]], bufs[b], sems[b])

    def gather_wait(j, b):
        pltpu.make_async_copy(hp_hbm.at[rowv.at[j]], bufs[b], sems[b]).wait()

    def scatter(j, b):
        pltpu.async_copy(bufs[b], acc.at[colv.at[j]], sems[b], add=True)

    def scatter_wait(b):
        pltpu.make_async_copy(bufs[b], acc.at[colv.at[0]], sems[b]).wait()

    for ph in range(PHASES):
        pltpu.sync_copy(row3_hbm.at[w * PHASES + ph], rowv)
        pltpu.sync_copy(col3_hbm.at[w * PHASES + ph], colv)
        gather(0, 0)
        gather(1, 1)

        def group(g, carry):
            for b in range(NB):
                j = g * NB + b
                gather_wait(j, b)
                scatter(j, b)
                b2 = (b + 2) % NB

                @pl.when(j >= 2)
                def _():
                    scatter_wait(b2)

                @pl.when(j + 2 < PCH)
                def _():
                    gather(j + 2, b2)
            return carry

        lax.fori_loop(0, PCH // NB, group, 0)
        scatter_wait((PCH - 2) % NB)
        scatter_wait((PCH - 1) % NB)
    plsc.subcore_barrier()

    @pl.when(s < NS - 1)
    def _():
        pltpu.sync_copy(acc.at[pl.ds(r0, RPT)],
                        out_hbm.at[c, pl.ds(r0, RPT)])

    @pl.when(s == NS - 1)
    def _():
        pltpu.sync_copy(acc.at[pl.ds((NS - 1) * RPT, RPT_LAST)],
                        out_hbm.at[c, pl.ds((NS - 1) * RPT, RPT_LAST)])


@functools.lru_cache(maxsize=None)
def _sc_kernels():
    mesh = plsc.VectorSubcoreMesh(
        core_axis_name="c", subcore_axis_name="s",
        num_cores=NC, num_subcores=NS)
    deg = pl.kernel(
        _deg_body,
        out_type=jax.ShapeDtypeStruct((NC, N), jnp.float32),
        mesh=mesh,
        scratch_types=[
            pltpu.VMEM((NCHD, CHD), jnp.int32),
            pltpu.VMEM((128,), jnp.float32),
            pltpu.VMEM_SHARED((N,), jnp.float32),
        ],
    )
    prop = pl.kernel(
        _prop_body,
        out_type=jax.ShapeDtypeStruct((NC, N, HH), jnp.float32),
        mesh=mesh,
        scratch_types=(
            [pltpu.VMEM((PCH, CH), jnp.int32),
             pltpu.VMEM((PCH, CH), jnp.int32),
             pltpu.VMEM((NB * CH, HH), jnp.float32),
             pltpu.VMEM_SHARED((N, HH), jnp.float32)]
            + [pltpu.SemaphoreType.DMA] * NB
        ),
        compiler_params=pltpu.CompilerParams(use_tc_tiling_on_sc=False),
    )
    return deg, prop


def _rsqrt(v):
    r = lax.rsqrt(v)
    return r * (1.5 - 0.5 * v * r * r)


def _recip(b):
    q = 1.0 / b
    return q * (2.0 - b * q)


def _exp(x):
    t = jnp.clip(x * 1.4426950408889634, -126.0, 126.0)
    k = jnp.round(t)
    u = (t - k) * 0.6931471805599453
    p = 1.0 + u * (1.0 + u * (0.5 + u * (
        1.0 / 6.0 + u * (1.0 / 24.0 + u * (1.0 / 120.0 + u * (1.0 / 720.0))))))
    ki = k.astype(jnp.int32)
    scale = lax.bitcast_convert_type((ki + 127) << 23, jnp.float32)
    return p * scale


def _bn_body(x_ref, g_ref, b_ref, w_ref, t_ref):
    x = x_ref[...]
    mu = jnp.mean(x, axis=0, keepdims=True)
    xc = x - mu
    var = jnp.mean(xc * xc, axis=0, keepdims=True)
    bn = xc * _rsqrt(var + 1e-5) * g_ref[...] + b_ref[...]
    t_ref[...] = jnp.dot(bn, w_ref[...], preferred_element_type=jnp.float32, precision=lax.Precision.HIGHEST)


def _hp_body(t_ref, degT_ref, dinv_ref, hp2_ref):
    deg = degT_ref[:, 0:1] + degT_ref[:, 1:2] + 1.0
    dinv = _rsqrt(deg)
    dinv_ref[...] = dinv
    hp = t_ref[...] * dinv
    hp2_ref[0] = hp[:, :HH]
    hp2_ref[1] = hp[:, HH:]


def _combine_body(p_ref, hp2_ref, dinv_ref, b_ref, w_ref, hpn_ref):
    dinv = dinv_ref[...]
    scat = jnp.concatenate([p_ref[0] + hp2_ref[0], p_ref[1] + hp2_ref[1]],
                           axis=1)
    pre = dinv * scat + b_ref[...]
    hin = jnp.maximum(pre, 0.0)
    t = jnp.dot(hin, w_ref[...], preferred_element_type=jnp.float32, precision=lax.Precision.HIGHEST)
    hp = t * dinv
    hpn_ref[0] = hp[:, :HH]
    hpn_ref[1] = hp[:, HH:]


def _tail_body(p_ref, hp2_ref, dinv_ref, b_ref, batch_ref,
               wd_ref, zb_ref, gc_ref, bc_ref, wc_ref, cc_ref,
               go_ref, bo_ref, wo_ref, co_ref, outc_ref, outo_ref):
    dinv = dinv_ref[...]
    scat = jnp.concatenate([p_ref[0] + hp2_ref[0], p_ref[1] + hp2_ref[1]],
                           axis=1)
    pre = dinv * scat + b_ref[...]
    h = jnp.maximum(pre, 0.0)
    z = jnp.dot(h, wd_ref[...], preferred_element_type=jnp.float32, precision=lax.Precision.HIGHEST) + zb_ref[...]
    a0 = _recip(1.0 + _exp(-z))
    xw = jnp.concatenate([a0 * h, (1.0 - a0) * h], axis=1)
    onehotT = (batch_ref[...] ==
               lax.broadcasted_iota(jnp.int32, (NG, 1), 0)).astype(jnp.float32)
    pooled = jnp.dot(onehotT, xw, preferred_element_type=jnp.float32, precision=lax.Precision.HIGHEST)
    cnt = jnp.sum(onehotT, axis=1, keepdims=True)
    mean = pooled * _recip(jnp.maximum(cnt, 1.0))

    def bnorm(v, g, b):
        mu = jnp.mean(v, axis=0, keepdims=True)
        vc = v - mu
        var = jnp.mean(vc * vc, axis=0, keepdims=True)
        return vc * _rsqrt(var + 1e-5) * g + b

    mc = mean[:, :H]
    mo = mean[:, H:]
    outc_ref[...] = (jnp.dot(bnorm(mc, gc_ref[...], bc_ref[...]), wc_ref[...],
                             preferred_element_type=jnp.float32, precision=lax.Precision.HIGHEST) + cc_ref[...])
    outo_ref[...] = (jnp.dot(bnorm(mo, go_ref[...], bo_ref[...]), wo_ref[...],
                             preferred_element_type=jnp.float32, precision=lax.Precision.HIGHEST) + co_ref[...])


def _tc_call(body, out_shapes):
    return pl.pallas_call(
        body, out_shape=out_shapes,
        compiler_params=pltpu.CompilerParams(
            vmem_limit_bytes=100 * 1024 * 1024))


def kernel(x, edge_index, batch, g0, b0, W_feat, b_feat, W_conv0, b_conv0,
           W_conv1, b_conv1, W_conv2, b_conv2, W_edge, b_edge, W_node, b_node,
           g_ctx, b_ctx, W_ctx, c_ctx, g_obj, b_obj, W_obj, c_obj):
    rowcat = jnp.concatenate([edge_index[0], edge_index[0] + N])
    colcat = jnp.concatenate([edge_index[1], edge_index[1]])
    row4 = rowcat.reshape(NC * NS * PHASES, PCH, CH)
    col4 = colcat.reshape(NC * NS * PHASES, PCH, CH)
    col3 = edge_index[1].reshape(NW, NCHD, CHD)
    ones128 = jnp.ones((128,), jnp.float32)
    zeros1 = jnp.zeros((N,), jnp.float32)
    zerosNHH = jnp.zeros((N, HH), jnp.float32)
    batch_row = batch.reshape(1, N)

    _deg_kernel, _prop_kernel = _sc_kernels()
    degp = _deg_kernel(col3, ones128, zeros1)
    degT = degp.T

    t0 = _tc_call(_bn_body, jax.ShapeDtypeStruct((N, H), jnp.float32))(
        x, g0.reshape(1, F), b0.reshape(1, F), W_feat)

    dinv, hp2 = _tc_call(
        _hp_body, (jax.ShapeDtypeStruct((N, 1), jnp.float32),
                   jax.ShapeDtypeStruct((NC, N, HH), jnp.float32)))(t0, degT)

    biases = (b_feat, b_conv0, b_conv1, b_conv2)
    weights_next = (W_conv0, W_conv1, W_conv2)
    for l in range(3):
        p = _prop_kernel(hp2.reshape(NC * N, HH), row4, col4, zerosNHH)
        hp2 = _tc_call(_combine_body,
                       jax.ShapeDtypeStruct((NC, N, HH), jnp.float32))(
            p, hp2, dinv, biases[l].reshape(1, H), weights_next[l])

    p = _prop_kernel(hp2.reshape(NC * N, HH), row4, col4, zerosNHH)
    wd = (W_node[:, 0:1] - W_node[:, 1:2])
    zb = (b_node[0] - b_node[1]).reshape(1, 1)
    outc, outo = _tc_call(
        _tail_body, (jax.ShapeDtypeStruct((NG, C), jnp.float32),
                     jax.ShapeDtypeStruct((NG, C), jnp.float32)))(
        p, hp2, dinv, biases[3].reshape(1, H), batch_row,
        wd, zb, g_ctx.reshape(1, H), b_ctx.reshape(1, H), W_ctx,
        c_ctx.reshape(1, C), g_obj.reshape(1, H), b_obj.reshape(1, H), W_obj,
        c_obj.reshape(1, C))
    return (outc, outo)


# speedup vs baseline: 1.0639x; 1.0639x over previous
import functools

import jax
import jax.numpy as jnp
from jax import lax
from jax.experimental import pallas as pl
from jax.experimental.pallas import tpu as pltpu
from jax.experimental.pallas import tpu_sc as plsc

N = 10000
E = 320000
F = 128
H = 128
C = 10
NG = 128

NC = 2
NS = 16
NW = NC * NS
CHD = 100
NCHD = E // (NW * CHD)
CH = 50
NB = 4
PHASES = 4
PCH = E // (NW * PHASES * CH)
RPT = 624
RPT_LAST = N - (NS - 1) * RPT


def _deg_body(col3_hbm, ones_hbm, zeros1_hbm, out_hbm, colv, onesv, acc1):
    c = lax.axis_index("c")
    s = lax.axis_index("s")
    w = c * NS + s
    pltpu.sync_copy(col3_hbm.at[w], colv)
    pltpu.sync_copy(ones_hbm, onesv)

    @pl.when(s == 0)
    def _():
        pltpu.sync_copy(zeros1_hbm, acc1)

    plsc.subcore_barrier()

    def step(j, carry):
        pltpu.sync_copy(onesv.at[pl.ds(0, CHD)], acc1.at[colv.at[j]], add=True)
        return carry

    lax.fori_loop(0, NCHD, step, 0)
    plsc.subcore_barrier()

    @pl.when(s == 0)
    def _():
        pltpu.sync_copy(acc1, out_hbm.at[c])


def _prop_body(hp_hbm, row3_hbm, col3_hbm, zeros_hbm, out_hbm,
               rowv, colv, bufbig, acc, sem0, sem1, sem2, sem3):
    c = lax.axis_index("c")
    s = lax.axis_index("s")
    w = c * NS + s
    r0 = s * RPT

    @pl.when(s < NS - 1)
    def _():
        pltpu.sync_copy(zeros_hbm.at[pl.ds(r0, RPT)], acc.at[pl.ds(r0, RPT)])

    @pl.when(s == NS - 1)
    def _():
        pltpu.sync_copy(zeros_hbm.at[pl.ds((NS - 1) * RPT, RPT_LAST)],
                        acc.at[pl.ds((NS - 1) * RPT, RPT_LAST)])

    plsc.subcore_barrier()

    bufs = tuple(bufbig.at[pl.ds(b * CH, CH)] for b in range(NB))
    sems = (sem0, sem1, sem2, sem3)

    def gather(j, b):
        pltpu.async_copy(hp_hbm.at[rowv.at[j]], bufs[b], sems[b])

    def gather_wait(j, b):
        pltpu.make_async_copy(hp_hbm.at[rowv.at[j]], bufs[b], sems[b]).wait()

    def scatter(j, b):
        pltpu.async_copy(bufs[b], acc.at[colv.at[j]], sems[b], add=True)

    def scatter_wait(b):
        pltpu.make_async_copy(bufs[b], acc.at[colv.at[0]], sems[b]).wait()

    def chunk(j, b):
        gather_wait(j, b)
        scatter(j, b)
        b2 = (b + 2) % NB

        @pl.when(j >= 2)
        def _():
            scatter_wait(b2)

        @pl.when(j + 2 < PCH)
        def _():
            gather(j + 2, b2)

    def chunk_tail(j, b):
        gather_wait(j, b)
        scatter(j, b)
        scatter_wait((b + 2) % NB)

    for ph in range(PHASES):
        pltpu.sync_copy(row3_hbm.at[w * PHASES + ph], rowv)
        pltpu.sync_copy(col3_hbm.at[w * PHASES + ph], colv)
        gather(0, 0)
        gather(1, 1)

        def group(g, carry):
            for b in range(NB):
                chunk(g * NB + b, b)
            return carry

        lax.fori_loop(0, PCH // NB, group, 0)
        for j in range(PCH - PCH % NB, PCH):
            chunk_tail(j, j % NB)
        scatter_wait((PCH - 2) % NB)
        scatter_wait((PCH - 1) % NB)
    plsc.subcore_barrier()

    @pl.when(s < NS - 1)
    def _():
        pltpu.sync_copy(acc.at[pl.ds(r0, RPT)],
                        out_hbm.at[c, pl.ds(r0, RPT)])

    @pl.when(s == NS - 1)
    def _():
        pltpu.sync_copy(acc.at[pl.ds((NS - 1) * RPT, RPT_LAST)],
                        out_hbm.at[c, pl.ds((NS - 1) * RPT, RPT_LAST)])


@functools.lru_cache(maxsize=None)
def _sc_kernels():
    mesh = plsc.VectorSubcoreMesh(
        core_axis_name="c", subcore_axis_name="s",
        num_cores=NC, num_subcores=NS)
    deg = pl.kernel(
        _deg_body,
        out_type=jax.ShapeDtypeStruct((NC, N), jnp.float32),
        mesh=mesh,
        scratch_types=[
            pltpu.VMEM((NCHD, CHD), jnp.int32),
            pltpu.VMEM((128,), jnp.float32),
            pltpu.VMEM_SHARED((N,), jnp.float32),
        ],
    )
    prop = pl.kernel(
        _prop_body,
        out_type=jax.ShapeDtypeStruct((NC, N, H), jnp.float32),
        mesh=mesh,
        scratch_types=(
            [pltpu.VMEM((PCH, CH), jnp.int32),
             pltpu.VMEM((PCH, CH), jnp.int32),
             pltpu.VMEM((NB * CH, H), jnp.float32),
             pltpu.VMEM_SHARED((N, H), jnp.float32)]
            + [pltpu.SemaphoreType.DMA] * NB
        ),
    )
    return deg, prop


def _rsqrt(v):
    r = lax.rsqrt(v)
    return r * (1.5 - 0.5 * v * r * r)


def _recip(b):
    q = 1.0 / b
    return q * (2.0 - b * q)


def _exp(x):
    t = jnp.clip(x * 1.4426950408889634, -126.0, 126.0)
    k = jnp.round(t)
    u = (t - k) * 0.6931471805599453
    p = 1.0 + u * (1.0 + u * (0.5 + u * (
        1.0 / 6.0 + u * (1.0 / 24.0 + u * (1.0 / 120.0 + u * (1.0 / 720.0))))))
    ki = k.astype(jnp.int32)
    scale = lax.bitcast_convert_type((ki + 127) << 23, jnp.float32)
    return p * scale


def _dot(a, b):
    return jnp.dot(a, b, preferred_element_type=jnp.float32,
                   precision=lax.Precision.HIGHEST)


def _bnhp_body(x_ref, g_ref, b_ref, w_ref, degT_ref, dinv_ref, hp_ref):
    x = x_ref[...]
    mu = jnp.mean(x, axis=0, keepdims=True)
    xc = x - mu
    var = jnp.mean(xc * xc, axis=0, keepdims=True)
    bn = xc * _rsqrt(var + 1e-5) * g_ref[...] + b_ref[...]
    t = _dot(bn, w_ref[...])
    deg = degT_ref[:, 0:1] + degT_ref[:, 1:2] + 1.0
    dinv = _rsqrt(deg)
    dinv_ref[...] = dinv
    hp_ref[...] = t * dinv


def _combine_body(p_ref, hp_ref, dinv_ref, b_ref, w_ref, hpn_ref):
    dinv = dinv_ref[...]
    pre = dinv * (p_ref[0] + p_ref[1] + hp_ref[...]) + b_ref[...]
    hin = jnp.maximum(pre, 0.0)
    t = _dot(hin, w_ref[...])
    hpn_ref[...] = t * dinv


def _tail_body(p_ref, hp_ref, dinv_ref, b_ref, batch_ref,
               wd_ref, zb_ref, gc_ref, bc_ref, wc_ref, cc_ref,
               go_ref, bo_ref, wo_ref, co_ref, outc_ref, outo_ref):
    dinv = dinv_ref[...]
    pre = dinv * (p_ref[0] + p_ref[1] + hp_ref[...]) + b_ref[...]
    h = jnp.maximum(pre, 0.0)
    z = _dot(h, wd_ref[...]) + zb_ref[...]
    a0 = _recip(1.0 + _exp(-z))
    xw = jnp.concatenate([a0 * h, (1.0 - a0) * h], axis=1)
    onehotT = (batch_ref[...] ==
               lax.broadcasted_iota(jnp.int32, (NG, 1), 0)).astype(jnp.float32)
    pooled = _dot(onehotT, xw)
    cnt = jnp.sum(onehotT, axis=1, keepdims=True)
    mean = pooled * _recip(jnp.maximum(cnt, 1.0))

    def bnorm(v, g, b):
        mu = jnp.mean(v, axis=0, keepdims=True)
        vc = v - mu
        var = jnp.mean(vc * vc, axis=0, keepdims=True)
        return vc * _rsqrt(var + 1e-5) * g + b

    mc = mean[:, :H]
    mo = mean[:, H:]
    outc_ref[...] = _dot(bnorm(mc, gc_ref[...], bc_ref[...]),
                         wc_ref[...]) + cc_ref[...]
    outo_ref[...] = _dot(bnorm(mo, go_ref[...], bo_ref[...]),
                         wo_ref[...]) + co_ref[...]


def _tc_call(body, out_shapes):
    return pl.pallas_call(
        body, out_shape=out_shapes,
        compiler_params=pltpu.CompilerParams(
            vmem_limit_bytes=100 * 1024 * 1024))


def kernel(x, edge_index, batch, g0, b0, W_feat, b_feat, W_conv0, b_conv0,
           W_conv1, b_conv1, W_conv2, b_conv2, W_edge, b_edge, W_node, b_node,
           g_ctx, b_ctx, W_ctx, c_ctx, g_obj, b_obj, W_obj, c_obj):
    row4 = edge_index[0].reshape(NW * PHASES, PCH, CH)
    col4 = edge_index[1].reshape(NW * PHASES, PCH, CH)
    col3 = edge_index[1].reshape(NW, NCHD, CHD)
    ones128 = jnp.ones((128,), jnp.float32)
    zeros1 = jnp.zeros((N,), jnp.float32)
    zerosNH = jnp.zeros((N, H), jnp.float32)
    batch_row = batch.reshape(1, N)

    _deg_kernel, _prop_kernel = _sc_kernels()
    degp = _deg_kernel(col3, ones128, zeros1)
    degT = degp.T

    dinv, hp = _tc_call(
        _bnhp_body, (jax.ShapeDtypeStruct((N, 1), jnp.float32),
                     jax.ShapeDtypeStruct((N, H), jnp.float32)))(
        x, g0.reshape(1, F), b0.reshape(1, F), W_feat, degT)

    biases = (b_feat, b_conv0, b_conv1, b_conv2)
    weights_next = (W_conv0, W_conv1, W_conv2)
    for l in range(3):
        p = _prop_kernel(hp, row4, col4, zerosNH)
        hp = _tc_call(_combine_body,
                      jax.ShapeDtypeStruct((N, H), jnp.float32))(
            p, hp, dinv, biases[l].reshape(1, H), weights_next[l])

    p = _prop_kernel(hp, row4, col4, zerosNH)
    wd = (W_node[:, 0:1] - W_node[:, 1:2])
    zb = (b_node[0] - b_node[1]).reshape(1, 1)
    outc, outo = _tc_call(
        _tail_body, (jax.ShapeDtypeStruct((NG, C), jnp.float32),
                     jax.ShapeDtypeStruct((NG, C), jnp.float32)))(
        p, hp, dinv, biases[3].reshape(1, H), batch_row,
        wd, zb, g_ctx.reshape(1, H), b_ctx.reshape(1, H), W_ctx,
        c_ctx.reshape(1, C), g_obj.reshape(1, H), b_obj.reshape(1, H), W_obj,
        c_obj.reshape(1, C))
    return (outc, outo)


# speedup vs baseline: 1.2451x; 1.1703x over previous
import functools

import jax
import jax.numpy as jnp
from jax import lax
from jax.experimental import pallas as pl
from jax.experimental.pallas import tpu as pltpu
from jax.experimental.pallas import tpu_sc as plsc

N = 10000
E = 320000
F = 128
H = 128
C = 10
NG = 128

NC = 2
NS = 16
NW = NC * NS
CHD = 100
NCHD = E // (NW * CHD)
CH = 100
NB = 3
PHASES = 4
PCH = E // (NW * PHASES * CH)
RPT = 624
RPT_LAST = N - (NS - 1) * RPT


def _deg_body(col3_hbm, ones_hbm, zeros1_hbm, out_hbm, colv, onesv, acc1):
    c = lax.axis_index("c")
    s = lax.axis_index("s")
    w = c * NS + s
    pltpu.sync_copy(col3_hbm.at[w], colv)
    pltpu.sync_copy(ones_hbm, onesv)

    @pl.when(s == 0)
    def _():
        pltpu.sync_copy(zeros1_hbm, acc1)

    plsc.subcore_barrier()

    def step(j, carry):
        pltpu.sync_copy(onesv.at[pl.ds(0, CHD)], acc1.at[colv.at[j]], add=True)
        return carry

    lax.fori_loop(0, NCHD, step, 0)
    plsc.subcore_barrier()

    @pl.when(s == 0)
    def _():
        pltpu.sync_copy(acc1, out_hbm.at[c])


def _prop_body(hp_hbm, row3_hbm, col3_hbm, zeros_hbm, out_hbm,
               rowv, colv, bufbig, acc, sem0, sem1, sem2, sem3):
    c = lax.axis_index("c")
    s = lax.axis_index("s")
    w = c * NS + s
    r0 = s * RPT

    @pl.when(s < NS - 1)
    def _():
        pltpu.sync_copy(zeros_hbm.at[pl.ds(r0, RPT)], acc.at[pl.ds(r0, RPT)])

    @pl.when(s == NS - 1)
    def _():
        pltpu.sync_copy(zeros_hbm.at[pl.ds((NS - 1) * RPT, RPT_LAST)],
                        acc.at[pl.ds((NS - 1) * RPT, RPT_LAST)])

    plsc.subcore_barrier()

    bufs = tuple(bufbig.at[pl.ds(b * CH, CH)] for b in range(NB))
    sems = (sem0, sem1, sem2, sem3)[:NB]

    def gather(j, b):
        pltpu.async_copy(hp_hbm.at[rowv.at[j]], bufs[b], sems[b])

    def gather_wait(j, b):
        pltpu.make_async_copy(hp_hbm.at[rowv.at[j]], bufs[b], sems[b]).wait()

    def scatter(j, b):
        pltpu.async_copy(bufs[b], acc.at[colv.at[j]], sems[b], add=True)

    def scatter_wait(b):
        pltpu.make_async_copy(bufs[b], acc.at[colv.at[0]], sems[b]).wait()

    def chunk(j, b):
        gather_wait(j, b)
        scatter(j, b)
        b2 = (b + 2) % NB

        @pl.when(j >= NB - 2)
        def _():
            scatter_wait(b2)

        @pl.when(j + 2 < PCH)
        def _():
            gather(j + 2, b2)

    def chunk_tail(j, b):
        gather_wait(j, b)
        scatter(j, b)
        scatter_wait((b + 2) % NB)

    for ph in range(PHASES):
        pltpu.sync_copy(row3_hbm.at[w * PHASES + ph], rowv)
        pltpu.sync_copy(col3_hbm.at[w * PHASES + ph], colv)
        gather(0, 0)
        gather(1, 1)

        def group(g, carry):
            for b in range(NB):
                chunk(g * NB + b, b)
            return carry

        lax.fori_loop(0, PCH // NB, group, 0)
        for j in range(PCH - PCH % NB, PCH):
            chunk_tail(j, j % NB)
        for i in range(NB - 2):
            scatter_wait((PCH - (NB - 2) + i) % NB)
    plsc.subcore_barrier()

    @pl.when(s < NS - 1)
    def _():
        pltpu.sync_copy(acc.at[pl.ds(r0, RPT)],
                        out_hbm.at[c, pl.ds(r0, RPT)])

    @pl.when(s == NS - 1)
    def _():
        pltpu.sync_copy(acc.at[pl.ds((NS - 1) * RPT, RPT_LAST)],
                        out_hbm.at[c, pl.ds((NS - 1) * RPT, RPT_LAST)])


@functools.lru_cache(maxsize=None)
def _sc_kernels():
    mesh = plsc.VectorSubcoreMesh(
        core_axis_name="c", subcore_axis_name="s",
        num_cores=NC, num_subcores=NS)
    deg = pl.kernel(
        _deg_body,
        out_type=jax.ShapeDtypeStruct((NC, N), jnp.float32),
        mesh=mesh,
        scratch_types=[
            pltpu.VMEM((NCHD, CHD), jnp.int32),
            pltpu.VMEM((128,), jnp.float32),
            pltpu.VMEM_SHARED((N,), jnp.float32),
        ],
    )
    prop = pl.kernel(
        _prop_body,
        out_type=jax.ShapeDtypeStruct((NC, N, H), jnp.float32),
        mesh=mesh,
        scratch_types=(
            [pltpu.VMEM((PCH, CH), jnp.int32),
             pltpu.VMEM((PCH, CH), jnp.int32),
             pltpu.VMEM((NB * CH, H), jnp.float32),
             pltpu.VMEM_SHARED((N, H), jnp.float32)]
            + [pltpu.SemaphoreType.DMA] * 4
        ),
    )
    return deg, prop


def _rsqrt(v):
    r = lax.rsqrt(v)
    return r * (1.5 - 0.5 * v * r * r)


def _recip(b):
    q = 1.0 / b
    return q * (2.0 - b * q)


def _exp(x):
    t = jnp.clip(x * 1.4426950408889634, -126.0, 126.0)
    k = jnp.round(t)
    u = (t - k) * 0.6931471805599453
    p = 1.0 + u * (1.0 + u * (0.5 + u * (
        1.0 / 6.0 + u * (1.0 / 24.0 + u * (1.0 / 120.0 + u * (1.0 / 720.0))))))
    ki = k.astype(jnp.int32)
    scale = lax.bitcast_convert_type((ki + 127) << 23, jnp.float32)
    return p * scale


def _dot(a, b):
    return jnp.dot(a, b, preferred_element_type=jnp.float32,
                   precision=lax.Precision.HIGHEST)


def _bnhp_body(x_ref, g_ref, b_ref, w_ref, degT_ref, dinv_ref, hp_ref):
    x = x_ref[...]
    mu = jnp.mean(x, axis=0, keepdims=True)
    xc = x - mu
    var = jnp.mean(xc * xc, axis=0, keepdims=True)
    bn = xc * _rsqrt(var + 1e-5) * g_ref[...] + b_ref[...]
    t = _dot(bn, w_ref[...])
    deg = degT_ref[:, 0:1] + degT_ref[:, 1:2] + 1.0
    dinv = _rsqrt(deg)
    dinv_ref[...] = dinv
    hp_ref[...] = t * dinv


def _combine_body(p_ref, hp_ref, dinv_ref, b_ref, w_ref, hpn_ref):
    dinv = dinv_ref[...]
    pre = dinv * (p_ref[0] + p_ref[1] + hp_ref[...]) + b_ref[...]
    hin = jnp.maximum(pre, 0.0)
    t = _dot(hin, w_ref[...])
    hpn_ref[...] = t * dinv


def _tail_body(p_ref, hp_ref, dinv_ref, b_ref, batch_ref,
               wd_ref, zb_ref, gc_ref, bc_ref, wc_ref, cc_ref,
               go_ref, bo_ref, wo_ref, co_ref, outc_ref, outo_ref):
    dinv = dinv_ref[...]
    pre = dinv * (p_ref[0] + p_ref[1] + hp_ref[...]) + b_ref[...]
    h = jnp.maximum(pre, 0.0)
    z = _dot(h, wd_ref[...]) + zb_ref[...]
    a0 = _recip(1.0 + _exp(-z))
    xw = jnp.concatenate([a0 * h, (1.0 - a0) * h], axis=1)
    onehotT = (batch_ref[...] ==
               lax.broadcasted_iota(jnp.int32, (NG, 1), 0)).astype(jnp.float32)
    pooled = _dot(onehotT, xw)
    cnt = jnp.sum(onehotT, axis=1, keepdims=True)
    mean = pooled * _recip(jnp.maximum(cnt, 1.0))

    def bnorm(v, g, b):
        mu = jnp.mean(v, axis=0, keepdims=True)
        vc = v - mu
        var = jnp.mean(vc * vc, axis=0, keepdims=True)
        return vc * _rsqrt(var + 1e-5) * g + b

    mc = mean[:, :H]
    mo = mean[:, H:]
    outc_ref[...] = _dot(bnorm(mc, gc_ref[...], bc_ref[...]),
                         wc_ref[...]) + cc_ref[...]
    outo_ref[...] = _dot(bnorm(mo, go_ref[...], bo_ref[...]),
                         wo_ref[...]) + co_ref[...]


def _tc_call(body, out_shapes):
    return pl.pallas_call(
        body, out_shape=out_shapes,
        compiler_params=pltpu.CompilerParams(
            vmem_limit_bytes=100 * 1024 * 1024))


def kernel(x, edge_index, batch, g0, b0, W_feat, b_feat, W_conv0, b_conv0,
           W_conv1, b_conv1, W_conv2, b_conv2, W_edge, b_edge, W_node, b_node,
           g_ctx, b_ctx, W_ctx, c_ctx, g_obj, b_obj, W_obj, c_obj):
    row4 = edge_index[0].reshape(NW * PHASES, PCH, CH)
    col4 = edge_index[1].reshape(NW * PHASES, PCH, CH)
    col3 = edge_index[1].reshape(NW, NCHD, CHD)
    ones128 = jnp.ones((128,), jnp.float32)
    zeros1 = jnp.zeros((N,), jnp.float32)
    zerosNH = jnp.zeros((N, H), jnp.float32)
    batch_row = batch.reshape(1, N)

    _deg_kernel, _prop_kernel = _sc_kernels()
    degp = _deg_kernel(col3, ones128, zeros1)
    degT = degp.T

    dinv, hp = _tc_call(
        _bnhp_body, (jax.ShapeDtypeStruct((N, 1), jnp.float32),
                     jax.ShapeDtypeStruct((N, H), jnp.float32)))(
        x, g0.reshape(1, F), b0.reshape(1, F), W_feat, degT)

    biases = (b_feat, b_conv0, b_conv1, b_conv2)
    weights_next = (W_conv0, W_conv1, W_conv2)
    for l in range(3):
        p = _prop_kernel(hp, row4, col4, zerosNH)
        hp = _tc_call(_combine_body,
                      jax.ShapeDtypeStruct((N, H), jnp.float32))(
            p, hp, dinv, biases[l].reshape(1, H), weights_next[l])

    p = _prop_kernel(hp, row4, col4, zerosNH)
    wd = (W_node[:, 0:1] - W_node[:, 1:2])
    zb = (b_node[0] - b_node[1]).reshape(1, 1)
    outc, outo = _tc_call(
        _tail_body, (jax.ShapeDtypeStruct((NG, C), jnp.float32),
                     jax.ShapeDtypeStruct((NG, C), jnp.float32)))(
        p, hp, dinv, biases[3].reshape(1, H), batch_row,
        wd, zb, g_ctx.reshape(1, H), b_ctx.reshape(1, H), W_ctx,
        c_ctx.reshape(1, C), g_obj.reshape(1, H), b_obj.reshape(1, H), W_obj,
        c_obj.reshape(1, C))
    return (outc, outo)


# speedup vs baseline: 1.2698x; 1.0198x over previous
import functools

import jax
import jax.numpy as jnp
from jax import lax
from jax.experimental import pallas as pl
from jax.experimental.pallas import tpu as pltpu
from jax.experimental.pallas import tpu_sc as plsc

N = 10000
E = 320000
F = 128
H = 128
C = 10
NG = 128

NC = 2
NS = 16
NW = NC * NS
CHD = 100
NCHD = E // (NW * CHD)
CH = 100
NB = 3
PHASES = 4
PCH = E // (NW * PHASES * CH)
RPT = 624
RPT_LAST = N - (NS - 1) * RPT


def _deg_body(col3_hbm, ones_hbm, zeros1_hbm, out_hbm, colv, onesv, acc1):
    c = lax.axis_index("c")
    s = lax.axis_index("s")
    w = c * NS + s
    pltpu.sync_copy(col3_hbm.at[w], colv)
    pltpu.sync_copy(ones_hbm, onesv)

    @pl.when(s == 0)
    def _():
        pltpu.sync_copy(zeros1_hbm, acc1)

    plsc.subcore_barrier()

    def step(j, carry):
        pltpu.sync_copy(onesv.at[pl.ds(0, CHD)], acc1.at[colv.at[j]], add=True)
        return carry

    lax.fori_loop(0, NCHD, step, 0)
    plsc.subcore_barrier()

    @pl.when(s == 0)
    def _():
        pltpu.sync_copy(acc1, out_hbm.at[c])


def _prop_body(hp_hbm, row3_hbm, col3_hbm, zeros_hbm, out_hbm,
               rowv, colv, bufbig, acc, sem0, sem1, sem2, sem3):
    c = lax.axis_index("c")
    s = lax.axis_index("s")
    w = c * NS + s
    r0 = s * RPT

    def init(src_hbm):
        @pl.when(s < NS - 1)
        def _():
            pltpu.sync_copy(src_hbm.at[pl.ds(r0, RPT)],
                            acc.at[pl.ds(r0, RPT)])

        @pl.when(s == NS - 1)
        def _():
            pltpu.sync_copy(src_hbm.at[pl.ds((NS - 1) * RPT, RPT_LAST)],
                            acc.at[pl.ds((NS - 1) * RPT, RPT_LAST)])

    @pl.when(c == 0)
    def _():
        init(hp_hbm)

    @pl.when(c == 1)
    def _():
        init(zeros_hbm)

    plsc.subcore_barrier()

    bufs = tuple(bufbig.at[pl.ds(b * CH, CH)] for b in range(NB))
    sems = (sem0, sem1, sem2, sem3)[:NB]

    def gather(j, b):
        pltpu.async_copy(hp_hbm.at[rowv.at[j]], bufs[b], sems[b])

    def gather_wait(j, b):
        pltpu.make_async_copy(hp_hbm.at[rowv.at[j]], bufs[b], sems[b]).wait()

    def scatter(j, b):
        pltpu.async_copy(bufs[b], acc.at[colv.at[j]], sems[b], add=True)

    def scatter_wait(b):
        pltpu.make_async_copy(bufs[b], acc.at[colv.at[0]], sems[b]).wait()

    def chunk(j, b):
        gather_wait(j, b)
        scatter(j, b)
        b2 = (b + 2) % NB

        @pl.when(j >= NB - 2)
        def _():
            scatter_wait(b2)

        @pl.when(j + 2 < PCH)
        def _():
            gather(j + 2, b2)

    def chunk_tail(j, b):
        gather_wait(j, b)
        scatter(j, b)
        scatter_wait((b + 2) % NB)

    for ph in range(PHASES):
        pltpu.sync_copy(row3_hbm.at[w * PHASES + ph], rowv)
        pltpu.sync_copy(col3_hbm.at[w * PHASES + ph], colv)
        gather(0, 0)
        gather(1, 1)

        def group(g, carry):
            for b in range(NB):
                chunk(g * NB + b, b)
            return carry

        lax.fori_loop(0, PCH // NB, group, 0)
        for j in range(PCH - PCH % NB, PCH):
            chunk_tail(j, j % NB)
        for i in range(NB - 2):
            scatter_wait((PCH - (NB - 2) + i) % NB)
    plsc.subcore_barrier()

    @pl.when(s < NS - 1)
    def _():
        pltpu.sync_copy(acc.at[pl.ds(r0, RPT)],
                        out_hbm.at[c, pl.ds(r0, RPT)])

    @pl.when(s == NS - 1)
    def _():
        pltpu.sync_copy(acc.at[pl.ds((NS - 1) * RPT, RPT_LAST)],
                        out_hbm.at[c, pl.ds((NS - 1) * RPT, RPT_LAST)])


@functools.lru_cache(maxsize=None)
def _sc_kernels():
    mesh = plsc.VectorSubcoreMesh(
        core_axis_name="c", subcore_axis_name="s",
        num_cores=NC, num_subcores=NS)
    deg = pl.kernel(
        _deg_body,
        out_type=jax.ShapeDtypeStruct((NC, N), jnp.float32),
        mesh=mesh,
        scratch_types=[
            pltpu.VMEM((NCHD, CHD), jnp.int32),
            pltpu.VMEM((128,), jnp.float32),
            pltpu.VMEM_SHARED((N,), jnp.float32),
        ],
    )
    prop = pl.kernel(
        _prop_body,
        out_type=jax.ShapeDtypeStruct((NC, N, H), jnp.float32),
        mesh=mesh,
        scratch_types=(
            [pltpu.VMEM((PCH, CH), jnp.int32),
             pltpu.VMEM((PCH, CH), jnp.int32),
             pltpu.VMEM((NB * CH, H), jnp.float32),
             pltpu.VMEM_SHARED((N, H), jnp.float32)]
            + [pltpu.SemaphoreType.DMA] * 4
        ),
    )
    return deg, prop


def _rsqrt(v):
    r = lax.rsqrt(v)
    return r * (1.5 - 0.5 * v * r * r)


def _recip(b):
    q = 1.0 / b
    return q * (2.0 - b * q)


def _exp(x):
    t = jnp.clip(x * 1.4426950408889634, -126.0, 126.0)
    k = jnp.round(t)
    u = (t - k) * 0.6931471805599453
    p = 1.0 + u * (1.0 + u * (0.5 + u * (
        1.0 / 6.0 + u * (1.0 / 24.0 + u * (1.0 / 120.0 + u * (1.0 / 720.0))))))
    ki = k.astype(jnp.int32)
    scale = lax.bitcast_convert_type((ki + 127) << 23, jnp.float32)
    return p * scale


def _dot(a, b):
    return jnp.dot(a, b, preferred_element_type=jnp.float32,
                   precision=lax.Precision.HIGHEST)


def _bn_body(x_ref, g_ref, b_ref, w_ref, t_ref):
    x = x_ref[...]
    mu = jnp.mean(x, axis=0, keepdims=True)
    xc = x - mu
    var = jnp.mean(xc * xc, axis=0, keepdims=True)
    bn = xc * _rsqrt(var + 1e-5) * g_ref[...] + b_ref[...]
    t_ref[...] = _dot(bn, w_ref[...])


def _hp_body(t_ref, degT_ref, dinv_ref, hp_ref):
    deg = degT_ref[:, 0:1] + degT_ref[:, 1:2] + 1.0
    dinv = _rsqrt(deg)
    dinv_ref[...] = dinv
    hp_ref[...] = t_ref[...] * dinv


def _combine_body(p_ref, dinv_ref, b_ref, w_ref, hpn_ref):
    dinv = dinv_ref[...]
    pre = dinv * (p_ref[0] + p_ref[1]) + b_ref[...]
    hin = jnp.maximum(pre, 0.0)
    t = _dot(hin, w_ref[...])
    hpn_ref[...] = t * dinv


def _tail_body(p_ref, dinv_ref, b_ref, batch_ref,
               wd_ref, zb_ref, gc_ref, bc_ref, wc_ref, cc_ref,
               go_ref, bo_ref, wo_ref, co_ref, outc_ref, outo_ref):
    dinv = dinv_ref[...]
    pre = dinv * (p_ref[0] + p_ref[1]) + b_ref[...]
    h = jnp.maximum(pre, 0.0)
    z = _dot(h, wd_ref[...]) + zb_ref[...]
    a0 = _recip(1.0 + _exp(-z))
    xw = jnp.concatenate([a0 * h, (1.0 - a0) * h], axis=1)
    onehotT = (batch_ref[...] ==
               lax.broadcasted_iota(jnp.int32, (NG, 1), 0)).astype(jnp.float32)
    pooled = _dot(onehotT, xw)
    cnt = jnp.sum(onehotT, axis=1, keepdims=True)
    mean = pooled * _recip(jnp.maximum(cnt, 1.0))

    def bnorm(v, g, b):
        mu = jnp.mean(v, axis=0, keepdims=True)
        vc = v - mu
        var = jnp.mean(vc * vc, axis=0, keepdims=True)
        return vc * _rsqrt(var + 1e-5) * g + b

    mc = mean[:, :H]
    mo = mean[:, H:]
    outc_ref[...] = _dot(bnorm(mc, gc_ref[...], bc_ref[...]),
                         wc_ref[...]) + cc_ref[...]
    outo_ref[...] = _dot(bnorm(mo, go_ref[...], bo_ref[...]),
                         wo_ref[...]) + co_ref[...]


def _tc_call(body, out_shapes):
    return pl.pallas_call(
        body, out_shape=out_shapes,
        compiler_params=pltpu.CompilerParams(
            vmem_limit_bytes=100 * 1024 * 1024))


def kernel(x, edge_index, batch, g0, b0, W_feat, b_feat, W_conv0, b_conv0,
           W_conv1, b_conv1, W_conv2, b_conv2, W_edge, b_edge, W_node, b_node,
           g_ctx, b_ctx, W_ctx, c_ctx, g_obj, b_obj, W_obj, c_obj):
    row4 = edge_index[0].reshape(NW * PHASES, PCH, CH)
    col4 = edge_index[1].reshape(NW * PHASES, PCH, CH)
    col3 = edge_index[1].reshape(NW, NCHD, CHD)
    ones128 = jnp.ones((128,), jnp.float32)
    zeros1 = jnp.zeros((N,), jnp.float32)
    zerosNH = jnp.zeros((N, H), jnp.float32)
    batch_row = batch.reshape(1, N)

    _deg_kernel, _prop_kernel = _sc_kernels()
    degp = _deg_kernel(col3, ones128, zeros1)
    degT = degp.T

    t0 = _tc_call(_bn_body, jax.ShapeDtypeStruct((N, H), jnp.float32))(
        x, g0.reshape(1, F), b0.reshape(1, F), W_feat)
    dinv, hp = _tc_call(
        _hp_body, (jax.ShapeDtypeStruct((N, 1), jnp.float32),
                   jax.ShapeDtypeStruct((N, H), jnp.float32)))(t0, degT)

    biases = (b_feat, b_conv0, b_conv1, b_conv2)
    weights_next = (W_conv0, W_conv1, W_conv2)
    for l in range(3):
        p = _prop_kernel(hp, row4, col4, zerosNH)
        hp = _tc_call(_combine_body,
                      jax.ShapeDtypeStruct((N, H), jnp.float32))(
            p, dinv, biases[l].reshape(1, H), weights_next[l])

    p = _prop_kernel(hp, row4, col4, zerosNH)
    wd = (W_node[:, 0:1] - W_node[:, 1:2])
    zb = (b_node[0] - b_node[1]).reshape(1, 1)
    outc, outo = _tc_call(
        _tail_body, (jax.ShapeDtypeStruct((NG, C), jnp.float32),
                     jax.ShapeDtypeStruct((NG, C), jnp.float32)))(
        p, dinv, biases[3].reshape(1, H), batch_row,
        wd, zb, g_ctx.reshape(1, H), b_ctx.reshape(1, H), W_ctx,
        c_ctx.reshape(1, C), g_obj.reshape(1, H), b_obj.reshape(1, H), W_obj,
        c_obj.reshape(1, C))
    return (outc, outo)
